# baseline (speedup 1.0000x reference)
import functools
import pathlib

import jax
import jax.numpy as jnp
from jax import lax
from jax.experimental import pallas as pl
from jax.experimental.pallas import tpu as pltpu
from jax.experimental.pallas import tpu_sc as plsc

jax.config.update("jax_compilation_cache_dir",
                  str(pathlib.Path(__file__).resolve().parent / ".jax_cache"))
jax.config.update("jax_persistent_cache_min_compile_time_secs", 5.0)

N = 10000
E = 320000
K = E // 2

_NT = 16
_CHUNK = E // _NT
_LSEG = _CHUNK // 16
_GK = K // _NT


def _iota():
    return lax.iota(jnp.int32, 16)


def _splat(v):
    return _iota() * 0 + v


def _sort_body(keys_hbm, src_hbm, dst_hbm, out_hbm, h_k, h_p,
               chunk_k, chunk_p, st_idx, loc_k, loc_p, tmprow,
               hist, off, totals, lst, cnts, pub, locst, pubidx,
               tmp_a, tmp_b,
               g2):
    core = lax.axis_index("c")
    t = lax.axis_index("s")
    iota = _iota()
    ones = _splat(1)
    zeros = _splat(0)
    soff = core * E
    goff = core * 4096

    for r in range(2):
        for c in range(8):
            pubidx[r, pl.ds(c * 16, 16)] = (
                goff + ((r * 128 + c * 16) + iota) * 16 + t)

    for p_i in range(4):
        shift = 8 * p_i
        last = p_i == 3

        if p_i == 0:
            pltpu.sync_copy(keys_hbm.at[pl.ds(core * E + t * _CHUNK, _CHUNK)],
                            chunk_k.at[pl.ds(0, _CHUNK)])

        def zero_body(j, _):
            hist[pl.ds(j * 16, 16)] = zeros
            return 0
        lax.fori_loop(0, 256, zero_body, 0)

        def hist_body(i, _):
            g = iota * _LSEG + i
            k = plsc.load_gather(chunk_k, [g])
            d = lax.shift_right_logical(k, _splat(shift)) & _splat(255)
            plsc.addupdate_scatter(hist, [iota * 256 + d], ones)
            return 0
        lax.fori_loop(0, _LSEG, hist_body, 0)

        def tot_body(j, _):
            acc = zeros
            for l in range(16):
                acc = acc + hist[pl.ds(l * 256 + j * 16, 16)]
            totals[pl.ds(j * 16, 16)] = acc
            return 0
        lax.fori_loop(0, 16, tot_body, 0)

        tmp_a[...] = zeros

        def lst_body(j, _):
            v = totals[pl.ds(j * 16, 16)]
            carry = tmp_a[...]
            c = plsc.cumsum(v)
            lst[pl.ds(j * 16, 16)] = carry + c - v
            tmp_b[...] = carry + c
            tmp_a[...] = plsc.load_gather(tmp_b, [_splat(15)])
            return 0
        lax.fori_loop(0, 16, lst_body, 0)

        def lane_off_body(d, _):
            col = plsc.load_gather(hist, [iota * 256 + d])
            ex = plsc.cumsum(col) - col
            base = plsc.load_gather(lst, [_splat(d)])
            plsc.store_scatter(off, [iota * 256 + d], ex + base)
            return 0
        lax.fori_loop(0, 256, lane_off_body, 0)

        def perm_body(i, _):
            g = iota * _LSEG + i
            k = plsc.load_gather(chunk_k, [g])
            if p_i == 0:
                p = t * _CHUNK + g
            else:
                p = plsc.load_gather(chunk_p, [g])
            d = lax.shift_right_logical(k, _splat(shift)) & _splat(255)
            oi = iota * 256 + d
            dest = plsc.load_gather(off, [oi])
            plsc.addupdate_scatter(off, [oi], ones)
            plsc.store_scatter(loc_k, [dest], k)
            plsc.store_scatter(loc_p, [dest], p)
            return 0
        lax.fori_loop(0, _LSEG, perm_body, 0)

        pltpu.sync_copy(loc_k, h_k.at[pl.ds(soff + t * _CHUNK, _CHUNK)])
        pltpu.sync_copy(loc_p, h_p.at[pl.ds(soff + t * _CHUNK, _CHUNK)])
        for r in range(2):
            pltpu.sync_copy(totals.at[pl.ds(r * 128, 128)],
                            g2.at[pubidx.at[r]])
        plsc.subcore_barrier()

        pltpu.sync_copy(g2.at[pl.ds(goff, 4096)], cnts)
        tmp_a[...] = zeros

        def scan_body(j, _):
            v = cnts[pl.ds(j * 16, 16)]
            carry = tmp_a[...]
            c = plsc.cumsum(v)
            pub[pl.ds(j * 16, 16)] = carry + c - v
            tmp_b[...] = carry + c
            tmp_a[...] = plsc.load_gather(tmp_b, [_splat(15)])
            return 0
        lax.fori_loop(0, 256, scan_body, 0)

        tmp_a[...] = zeros

        def locst_body(j, _):
            v = cnts[pl.ds(j * 16, 16)]
            acc = tmp_a[...]
            locst[pl.ds(j * 16, 16)] = acc
            tmp_a[...] = acc + v
            return 0
        lax.fori_loop(0, 256, locst_body, 0)

        qlen = _GK if last else _CHUNK
        qbase = t * qlen

        tmp_a[...] = zeros

        def fb_body(j, _):
            gsv = pub[pl.ds(j * 16, 16)]
            nlo = plsc.all_reduce_population_count(gsv <= qbase)
            nhi = plsc.all_reduce_population_count(gsv < qbase + qlen)
            tmp_a[...] = tmp_a[...] + jnp.where(iota < 8, nlo, nhi)
            return 0
        lax.fori_loop(0, 256, fb_body, 0)
        fbv = tmp_a[...]
        f_lo = jnp.maximum(fbv[0] - 1, 0)
        f_hi = fbv[15]

        def seg_body(f, _):
            fvec = _splat(f)
            cntv = plsc.load_gather(cnts, [fvec])
            gsv = plsc.load_gather(pub, [fvec])
            lsv = plsc.load_gather(locst, [fvec])
            lov = jnp.maximum(gsv, _splat(qbase))
            hiv = jnp.minimum(gsv + cntv, _splat(qbase + qlen))
            nv = hiv - lov
            n = nv[0]

            @pl.when(n > 0)
            def _():
                srcbv = lsv + (lov - gsv) + (soff + (f & 15) * _CHUNK)
                dstbv = lov - _splat(qbase)

                def emit(j, _):
                    v = j * 16 + iota
                    s = dstbv + v
                    plsc.store_scatter(
                        st_idx,
                        [lax.shift_right_logical(s, _splat(7)),
                         s & _splat(127)],
                        srcbv + v, mask=v < nv)
                    return 0
                lax.fori_loop(0, (n + 15) // 16, emit, 0)
            return 0
        lax.fori_loop(f_lo, f_hi, seg_body, 0)

        if last:
            for c in range(1, 8):
                st_idx[_GK // 128, pl.ds(c * 16, 16)] = zeros
        else:
            for c in range(2, 8):
                st_idx[_CHUNK // 128, pl.ds(c * 16, 16)] = zeros

        nrows = _GK // 128 + 1 if last else _CHUNK // 128 + 1

        def gat_body(r, _):
            pltpu.sync_copy(h_p.at[st_idx.at[r]],
                            chunk_p.at[pl.ds(r * 128, 128)])
            if not last:
                pltpu.sync_copy(h_k.at[st_idx.at[r]],
                                chunk_k.at[pl.ds(r * 128, 128)])
            return 0
        lax.fori_loop(0, nrows, gat_body, 0)
        if not last:
            plsc.subcore_barrier()

    def extract_body(i, _):
        g = i * 16 + iota
        p = plsc.load_gather(chunk_p, [g])
        plsc.store_scatter(
            st_idx,
            [lax.shift_right_logical(g, _splat(7)), g & _splat(127)],
            p)
        return 0
    lax.fori_loop(0, _GK // 16, extract_body, 0)
    for c in range(1, 8):
        st_idx[_GK // 128, pl.ds(c * 16, 16)] = zeros

    obase = core * (2 * K) + t * _GK
    nfull = _GK // 128
    for ehbm, oo in ((src_hbm, 0), (dst_hbm, K)):
        def out_body(r, _):
            pltpu.sync_copy(ehbm.at[st_idx.at[r]], tmprow)
            pltpu.sync_copy(tmprow, out_hbm.at[pl.ds(obase + oo + r * 128,
                                                     128)])
            return 0
        lax.fori_loop(0, nfull, out_body, 0)
        pltpu.sync_copy(ehbm.at[st_idx.at[nfull]], tmprow)
        pltpu.sync_copy(tmprow.at[pl.ds(0, 16)],
                        out_hbm.at[pl.ds(obase + oo + nfull * 128, 16)])


def _sc_topk(keys2, srcv, dstv):
    mesh = plsc.VectorSubcoreMesh(core_axis_name="c", subcore_axis_name="s",
                                  num_cores=2, num_subcores=_NT)
    f = pl.kernel(
        _sort_body,
        out_type=[
            jax.ShapeDtypeStruct((2 * 2 * K,), jnp.int32),
            jax.ShapeDtypeStruct((2 * E,), jnp.int32),
            jax.ShapeDtypeStruct((2 * E,), jnp.int32),
        ],
        mesh=mesh,
        compiler_params=pltpu.CompilerParams(needs_layout_passes=False),
        scratch_types=[
            pltpu.VMEM((_CHUNK + 96,), jnp.int32),
            pltpu.VMEM((_CHUNK + 96,), jnp.int32),
            pltpu.VMEM((_CHUNK // 128 + 1, 128), jnp.int32),
            pltpu.VMEM((_CHUNK,), jnp.int32),
            pltpu.VMEM((_CHUNK,), jnp.int32),
            pltpu.VMEM((128,), jnp.int32),
            pltpu.VMEM((4096,), jnp.int32),
            pltpu.VMEM((4096,), jnp.int32),
            pltpu.VMEM((256,), jnp.int32),
            pltpu.VMEM((256,), jnp.int32),
            pltpu.VMEM((4096,), jnp.int32),
            pltpu.VMEM((4096,), jnp.int32),
            pltpu.VMEM((4096,), jnp.int32),
            pltpu.VMEM((2, 128), jnp.int32),
            pltpu.VMEM((16,), jnp.int32),
            pltpu.VMEM((16,), jnp.int32),
            pltpu.VMEM_SHARED((2 * 4096,), jnp.int32),
        ],
    )
    return f(keys2, srcv, dstv)[0]


def _gat_ref(x, edge_index, W, att_src, att_dst, b):
    n = x.shape[0]
    xp = x @ W
    loops = jnp.arange(n, dtype=edge_index.dtype)
    src = jnp.concatenate([edge_index[0], loops])
    dst = jnp.concatenate([edge_index[1], loops])
    a_src = (xp * att_src).sum(axis=-1)
    a_dst = (xp * att_dst).sum(axis=-1)
    alpha = jax.nn.leaky_relu(a_src[src] + a_dst[dst], negative_slope=0.2)
    amax = jax.ops.segment_max(alpha, dst, num_segments=n)
    amax = jnp.where(jnp.isfinite(amax), amax, 0.0)
    ex = jnp.exp(alpha - amax[dst])
    denom = jax.ops.segment_sum(ex, dst, num_segments=n)
    coef = ex / (denom[dst] + 1e-16)
    out = jax.ops.segment_sum(xp[src] * coef[:, None], dst, num_segments=n)
    return out + b


def kernel(x, edge_index, W1, att_src1, att_dst1, b1, W2, att_src2, att_dst2, b2):
    xM1 = jax.nn.leaky_relu(_gat_ref(x, edge_index, W1, att_src1, att_dst1, b1),
                            negative_slope=0.01)
    xM2 = _gat_ref(xM1, edge_index, W2, att_src2, att_dst2, b2)
    value = (xM2[edge_index[0]] * xM2[edge_index[1]]).sum(axis=1)

    bits = lax.bitcast_convert_type(value, jnp.uint32)
    sign = bits >> 31
    key_asc = bits ^ jnp.where(sign == 1, jnp.uint32(0xFFFFFFFF),
                               jnp.uint32(0x80000000))
    keys2 = jnp.concatenate([key_asc, ~key_asc]).astype(jnp.uint32)
    keys2 = lax.bitcast_convert_type(keys2, jnp.int32)

    out = _sc_topk(keys2, edge_index[0], edge_index[1]).reshape(2, 2, K)
    return (out[1], out[0], xM2)


# speedup vs baseline: 1.1541x; 1.1541x over previous
import functools
import pathlib

import jax
import jax.numpy as jnp
from jax import lax
from jax.experimental import pallas as pl
from jax.experimental.pallas import tpu as pltpu
from jax.experimental.pallas import tpu_sc as plsc

jax.config.update("jax_compilation_cache_dir",
                  str(pathlib.Path(__file__).resolve().parent / ".jax_cache"))
jax.config.update("jax_persistent_cache_min_compile_time_secs", 5.0)

N = 10000
E = 320000
K = E // 2

_NT = 16
_CHUNK = E // _NT
_LSEG = _CHUNK // 16
_GK = K // _NT


def _iota():
    return lax.iota(jnp.int32, 16)


def _splat(v):
    return _iota() * 0 + v


def _sort_body(keys_hbm, src_hbm, dst_hbm, out_hbm, h_k, h_p,
               chunk_k, chunk_p, st_idx, loc_k, loc_p, tmprow,
               hist, off, totals, lst, cnts, pub, locst, pubidx,
               tmp_a, tmp_b,
               g2, sem):
    core = lax.axis_index("c")
    t = lax.axis_index("s")
    iota = _iota()
    ones = _splat(1)
    zeros = _splat(0)
    soff = core * E
    goff = core * 4096

    for r in range(2):
        for c in range(8):
            pubidx[r, pl.ds(c * 16, 16)] = (
                goff + ((r * 128 + c * 16) + iota) * 16 + t)

    for rr in range(157, 160):
        for c in range(8):
            st_idx[rr, pl.ds(c * 16, 16)] = zeros

    for p_i in range(4):
        shift = 8 * p_i
        last = p_i == 3

        if p_i == 0:
            pltpu.sync_copy(keys_hbm.at[pl.ds(core * E + t * _CHUNK, _CHUNK)],
                            chunk_k.at[pl.ds(0, _CHUNK)])

        def zero_body(j, _):
            hist[pl.ds(j * 16, 16)] = zeros
            return 0
        lax.fori_loop(0, 256, zero_body, 0)

        def hist_body(i, _):
            g = iota * _LSEG + i
            k = plsc.load_gather(chunk_k, [g])
            d = lax.shift_right_logical(k, _splat(shift)) & _splat(255)
            plsc.addupdate_scatter(hist, [iota * 256 + d], ones)
            return 0
        lax.fori_loop(0, _LSEG, hist_body, 0)

        def tot_body(j, _):
            acc = zeros
            for l in range(16):
                acc = acc + hist[pl.ds(l * 256 + j * 16, 16)]
            totals[pl.ds(j * 16, 16)] = acc
            return 0
        lax.fori_loop(0, 16, tot_body, 0)

        tmp_a[...] = zeros

        def lst_body(j, _):
            v = totals[pl.ds(j * 16, 16)]
            carry = tmp_a[...]
            c = plsc.cumsum(v)
            lst[pl.ds(j * 16, 16)] = carry + c - v
            tmp_b[...] = carry + c
            tmp_a[...] = plsc.load_gather(tmp_b, [_splat(15)])
            return 0
        lax.fori_loop(0, 16, lst_body, 0)

        def lane_off_body(d, _):
            col = plsc.load_gather(hist, [iota * 256 + d])
            ex = plsc.cumsum(col) - col
            base = plsc.load_gather(lst, [_splat(d)])
            plsc.store_scatter(off, [iota * 256 + d], ex + base)
            return 0
        lax.fori_loop(0, 256, lane_off_body, 0)

        def perm_body(i, _):
            g = iota * _LSEG + i
            k = plsc.load_gather(chunk_k, [g])
            if p_i == 0:
                p = t * _CHUNK + g
            else:
                p = plsc.load_gather(chunk_p, [g])
            d = lax.shift_right_logical(k, _splat(shift)) & _splat(255)
            oi = iota * 256 + d
            dest = plsc.load_gather(off, [oi])
            plsc.addupdate_scatter(off, [oi], ones)
            plsc.store_scatter(loc_k, [dest], k)
            plsc.store_scatter(loc_p, [dest], p)
            return 0
        lax.fori_loop(0, _LSEG, perm_body, 0)

        pltpu.sync_copy(loc_k, h_k.at[pl.ds(soff + t * _CHUNK, _CHUNK)])
        pltpu.sync_copy(loc_p, h_p.at[pl.ds(soff + t * _CHUNK, _CHUNK)])
        for r in range(2):
            pltpu.sync_copy(totals.at[pl.ds(r * 128, 128)],
                            g2.at[pubidx.at[r]])
        plsc.subcore_barrier()

        pltpu.sync_copy(g2.at[pl.ds(goff, 4096)], cnts)
        tmp_a[...] = zeros

        def scan_body(j, _):
            v = cnts[pl.ds(j * 16, 16)]
            carry = tmp_a[...]
            c = plsc.cumsum(v)
            pub[pl.ds(j * 16, 16)] = carry + c - v
            tmp_b[...] = carry + c
            tmp_a[...] = plsc.load_gather(tmp_b, [_splat(15)])
            return 0
        lax.fori_loop(0, 256, scan_body, 0)

        tmp_a[...] = zeros

        def locst_body(j, _):
            v = cnts[pl.ds(j * 16, 16)]
            acc = tmp_a[...]
            locst[pl.ds(j * 16, 16)] = acc
            tmp_a[...] = acc + v
            return 0
        lax.fori_loop(0, 256, locst_body, 0)

        qlen = _GK if last else _CHUNK
        qbase = t * qlen

        tmp_a[...] = zeros

        def fb_body(j, _):
            gsv = pub[pl.ds(j * 16, 16)]
            nlo = plsc.all_reduce_population_count(gsv <= qbase)
            nhi = plsc.all_reduce_population_count(gsv < qbase + qlen)
            tmp_a[...] = tmp_a[...] + jnp.where(iota < 8, nlo, nhi)
            return 0
        lax.fori_loop(0, 256, fb_body, 0)
        fbv = tmp_a[...]
        f_lo = jnp.maximum(fbv[0] - 1, 0)
        f_hi = fbv[15]

        def seg_body(f, _):
            fvec = _splat(f)
            cntv = plsc.load_gather(cnts, [fvec])
            gsv = plsc.load_gather(pub, [fvec])
            lsv = plsc.load_gather(locst, [fvec])
            lov = jnp.maximum(gsv, _splat(qbase))
            hiv = jnp.minimum(gsv + cntv, _splat(qbase + qlen))
            nv = hiv - lov
            n = nv[0]

            @pl.when(n > 0)
            def _():
                srcbv = lsv + (lov - gsv) + (soff + (f & 15) * _CHUNK)
                dstbv = lov - _splat(qbase)

                def emit(j, _):
                    v = j * 16 + iota
                    s = dstbv + v
                    plsc.store_scatter(
                        st_idx,
                        [lax.shift_right_logical(s, _splat(7)),
                         s & _splat(127)],
                        srcbv + v, mask=v < nv)
                    return 0
                lax.fori_loop(0, (n + 15) // 16, emit, 0)
            return 0
        lax.fori_loop(f_lo, f_hi, seg_body, 0)

        if last:
            for c in range(1, 8):
                st_idx[_GK // 128, pl.ds(c * 16, 16)] = zeros
        else:
            for c in range(2, 8):
                st_idx[_CHUNK // 128, pl.ds(c * 16, 16)] = zeros

        ngrp = 10 if last else 20

        def gat_body(gi, _):
            descs = []
            for u in range(8):
                r = gi * 8 + u
                descs.append(pltpu.async_copy(
                    h_p.at[st_idx.at[r]],
                    chunk_p.at[pl.ds(r * 128, 128)], sem))
                if not last:
                    descs.append(pltpu.async_copy(
                        h_k.at[st_idx.at[r]],
                        chunk_k.at[pl.ds(r * 128, 128)], sem))
            for dsc in descs:
                dsc.wait()
            return 0
        lax.fori_loop(0, ngrp, gat_body, 0)
        if not last:
            plsc.subcore_barrier()

    def extract_body(i, _):
        g = i * 16 + iota
        p = plsc.load_gather(chunk_p, [g])
        plsc.store_scatter(
            st_idx,
            [lax.shift_right_logical(g, _splat(7)), g & _splat(127)],
            p)
        return 0
    lax.fori_loop(0, _GK // 16, extract_body, 0)
    for c in range(1, 8):
        st_idx[_GK // 128, pl.ds(c * 16, 16)] = zeros

    obase = core * (2 * K) + t * _GK
    for ehbm, oo in ((src_hbm, 0), (dst_hbm, K)):
        def _fire_group(gi):
            descs = [pltpu.async_copy(ehbm.at[st_idx.at[gi * 8 + u]],
                                      tmprow.at[pl.ds(u * 128, 128)], sem)
                     for u in range(8)]
            for dsc in descs:
                dsc.wait()

        def out_body(gi, _):
            _fire_group(gi)
            pltpu.sync_copy(tmprow,
                            out_hbm.at[pl.ds(obase + oo + gi * 1024, 1024)])
            return 0
        lax.fori_loop(0, 9, out_body, 0)
        _fire_group(9)
        pltpu.sync_copy(tmprow.at[pl.ds(0, 784)],
                        out_hbm.at[pl.ds(obase + oo + 9216, 784)])


def _sc_topk(keys2, srcv, dstv):
    mesh = plsc.VectorSubcoreMesh(core_axis_name="c", subcore_axis_name="s",
                                  num_cores=2, num_subcores=_NT)
    f = pl.kernel(
        _sort_body,
        out_type=[
            jax.ShapeDtypeStruct((2 * 2 * K,), jnp.int32),
            jax.ShapeDtypeStruct((2 * E,), jnp.int32),
            jax.ShapeDtypeStruct((2 * E,), jnp.int32),
        ],
        mesh=mesh,
        compiler_params=pltpu.CompilerParams(needs_layout_passes=False),
        scratch_types=[
            pltpu.VMEM((20480,), jnp.int32),
            pltpu.VMEM((20480,), jnp.int32),
            pltpu.VMEM((160, 128), jnp.int32),
            pltpu.VMEM((_CHUNK,), jnp.int32),
            pltpu.VMEM((_CHUNK,), jnp.int32),
            pltpu.VMEM((1024,), jnp.int32),
            pltpu.VMEM((4096,), jnp.int32),
            pltpu.VMEM((4096,), jnp.int32),
            pltpu.VMEM((256,), jnp.int32),
            pltpu.VMEM((256,), jnp.int32),
            pltpu.VMEM((4096,), jnp.int32),
            pltpu.VMEM((4096,), jnp.int32),
            pltpu.VMEM((4096,), jnp.int32),
            pltpu.VMEM((2, 128), jnp.int32),
            pltpu.VMEM((16,), jnp.int32),
            pltpu.VMEM((16,), jnp.int32),
            pltpu.VMEM_SHARED((2 * 4096,), jnp.int32),
            pltpu.SemaphoreType.DMA,
        ],
    )
    return f(keys2, srcv, dstv)[0]


def _gat_ref(x, edge_index, W, att_src, att_dst, b):
    n = x.shape[0]
    xp = x @ W
    loops = jnp.arange(n, dtype=edge_index.dtype)
    src = jnp.concatenate([edge_index[0], loops])
    dst = jnp.concatenate([edge_index[1], loops])
    a_src = (xp * att_src).sum(axis=-1)
    a_dst = (xp * att_dst).sum(axis=-1)
    alpha = jax.nn.leaky_relu(a_src[src] + a_dst[dst], negative_slope=0.2)
    amax = jax.ops.segment_max(alpha, dst, num_segments=n)
    amax = jnp.where(jnp.isfinite(amax), amax, 0.0)
    ex = jnp.exp(alpha - amax[dst])
    denom = jax.ops.segment_sum(ex, dst, num_segments=n)
    coef = ex / (denom[dst] + 1e-16)
    out = jax.ops.segment_sum(xp[src] * coef[:, None], dst, num_segments=n)
    return out + b


def _copy_kernel(x_ref, o_ref):
    o_ref[...] = x_ref[...]


def _pl_copy(x):
    return pl.pallas_call(
        _copy_kernel,
        out_shape=jax.ShapeDtypeStruct(x.shape, x.dtype),
    )(x)


def kernel(x, edge_index, W1, att_src1, att_dst1, b1, W2, att_src2, att_dst2, b2):
    x = _pl_copy(x)
    xM1 = jax.nn.leaky_relu(_gat_ref(x, edge_index, W1, att_src1, att_dst1, b1),
                            negative_slope=0.01)
    xM2 = _gat_ref(xM1, edge_index, W2, att_src2, att_dst2, b2)
    value = (xM2[edge_index[0]] * xM2[edge_index[1]]).sum(axis=1)

    bits = lax.bitcast_convert_type(value, jnp.uint32)
    sign = bits >> 31
    key_asc = bits ^ jnp.where(sign == 1, jnp.uint32(0xFFFFFFFF),
                               jnp.uint32(0x80000000))
    keys2 = jnp.concatenate([key_asc, ~key_asc]).astype(jnp.uint32)
    keys2 = lax.bitcast_convert_type(keys2, jnp.int32)

    out = _sc_topk(keys2, edge_index[0], edge_index[1]).reshape(2, 2, K)
    return (out[1], out[0], xM2)


# speedup vs baseline: 1.1666x; 1.0108x over previous
import functools
import pathlib

import jax
import jax.numpy as jnp
from jax import lax
from jax.experimental import pallas as pl
from jax.experimental.pallas import tpu as pltpu
from jax.experimental.pallas import tpu_sc as plsc

jax.config.update("jax_compilation_cache_dir",
                  str(pathlib.Path(__file__).resolve().parent / ".jax_cache"))
jax.config.update("jax_persistent_cache_min_compile_time_secs", 5.0)

N = 10000
E = 320000
K = E // 2

_NT = 16
_CHUNK = E // _NT
_LSEG = _CHUNK // 16
_GK = K // _NT


def _iota():
    return lax.iota(jnp.int32, 16)


def _splat(v):
    return _iota() * 0 + v


def _sort_body(keys_hbm, src_hbm, dst_hbm, out_hbm, h_k, h_p,
               chunk_k, chunk_p, st_idx, loc_k, loc_p, tmprow,
               hist, off, totals, lst, cnts, pub, locst, pubidx,
               tmp_a, tmp_b,
               g2, sem):
    core = lax.axis_index("c")
    t = lax.axis_index("s")
    iota = _iota()
    ones = _splat(1)
    zeros = _splat(0)
    soff = core * E
    goff = core * 4096

    for r in range(2):
        for c in range(8):
            pubidx[r, pl.ds(c * 16, 16)] = (
                goff + ((r * 128 + c * 16) + iota) * 16 + t)

    for rr in range(157, 160):
        for c in range(8):
            st_idx[rr, pl.ds(c * 16, 16)] = zeros

    for p_i in range(4):
        shift = 8 * p_i
        last = p_i == 3

        if p_i == 0:
            pltpu.sync_copy(keys_hbm.at[pl.ds(core * E + t * _CHUNK, _CHUNK)],
                            chunk_k.at[pl.ds(0, _CHUNK)])

        def zero_body(j, _):
            hist[pl.ds(j * 16, 16)] = zeros
            return 0
        lax.fori_loop(0, 256, zero_body, 0)

        def hist_body(i, _):
            g = iota * _LSEG + i
            k = plsc.load_gather(chunk_k, [g])
            d = lax.shift_right_logical(k, _splat(shift)) & _splat(255)
            plsc.addupdate_scatter(hist, [iota * 256 + d], ones)
            return 0
        lax.fori_loop(0, _LSEG, hist_body, 0)

        def tot_body(j, _):
            acc = zeros
            for l in range(16):
                acc = acc + hist[pl.ds(l * 256 + j * 16, 16)]
            totals[pl.ds(j * 16, 16)] = acc
            return 0
        lax.fori_loop(0, 16, tot_body, 0)

        tmp_a[...] = zeros

        def lst_body(j, _):
            v = totals[pl.ds(j * 16, 16)]
            carry = tmp_a[...]
            c = plsc.cumsum(v)
            lst[pl.ds(j * 16, 16)] = carry + c - v
            tmp_b[...] = carry + c
            tmp_a[...] = plsc.load_gather(tmp_b, [_splat(15)])
            return 0
        lax.fori_loop(0, 16, lst_body, 0)

        def lane_off_body(d, _):
            col = plsc.load_gather(hist, [iota * 256 + d])
            ex = plsc.cumsum(col) - col
            base = plsc.load_gather(lst, [_splat(d)])
            plsc.store_scatter(off, [iota * 256 + d], ex + base)
            return 0
        lax.fori_loop(0, 256, lane_off_body, 0)

        def perm_body(i, _):
            g = iota * _LSEG + i
            k = plsc.load_gather(chunk_k, [g])
            if p_i == 0:
                p = t * _CHUNK + g
            else:
                p = plsc.load_gather(chunk_p, [g])
            d = lax.shift_right_logical(k, _splat(shift)) & _splat(255)
            oi = iota * 256 + d
            dest = plsc.load_gather(off, [oi])
            plsc.addupdate_scatter(off, [oi], ones)
            plsc.store_scatter(loc_k, [dest], k)
            plsc.store_scatter(loc_p, [dest], p)
            return 0
        lax.fori_loop(0, _LSEG, perm_body, 0)

        pltpu.sync_copy(loc_k, h_k.at[pl.ds(soff + t * _CHUNK, _CHUNK)])
        pltpu.sync_copy(loc_p, h_p.at[pl.ds(soff + t * _CHUNK, _CHUNK)])
        for r in range(2):
            pltpu.sync_copy(totals.at[pl.ds(r * 128, 128)],
                            g2.at[pubidx.at[r]])
        plsc.subcore_barrier()

        pltpu.sync_copy(g2.at[pl.ds(goff, 4096)], cnts)
        tmp_a[...] = zeros

        def scan_body(j, _):
            v = cnts[pl.ds(j * 16, 16)]
            carry = tmp_a[...]
            c = plsc.cumsum(v)
            pub[pl.ds(j * 16, 16)] = carry + c - v
            tmp_b[...] = carry + c
            tmp_a[...] = plsc.load_gather(tmp_b, [_splat(15)])
            return 0
        lax.fori_loop(0, 256, scan_body, 0)

        tmp_a[...] = zeros

        def locst_body(j, _):
            v = cnts[pl.ds(j * 16, 16)]
            acc = tmp_a[...]
            locst[pl.ds(j * 16, 16)] = acc
            tmp_a[...] = acc + v
            return 0
        lax.fori_loop(0, 256, locst_body, 0)

        qlen = _GK if last else _CHUNK
        qbase = t * qlen

        tmp_a[...] = zeros

        def fb_body(j, _):
            gsv = pub[pl.ds(j * 16, 16)]
            nlo = plsc.all_reduce_population_count(gsv <= qbase)
            nhi = plsc.all_reduce_population_count(gsv < qbase + qlen)
            tmp_a[...] = tmp_a[...] + jnp.where(iota < 8, nlo, nhi)
            return 0
        lax.fori_loop(0, 256, fb_body, 0)
        fbv = tmp_a[...]
        f_lo = jnp.maximum(fbv[0] - 1, 0)
        f_hi = fbv[15]

        def seg_body(f, _):
            fvec = _splat(f)
            cntv = plsc.load_gather(cnts, [fvec])
            gsv = plsc.load_gather(pub, [fvec])
            lsv = plsc.load_gather(locst, [fvec])
            lov = jnp.maximum(gsv, _splat(qbase))
            hiv = jnp.minimum(gsv + cntv, _splat(qbase + qlen))
            nv = hiv - lov
            n = nv[0]

            @pl.when(n > 0)
            def _():
                srcbv = lsv + (lov - gsv) + (soff + (f & 15) * _CHUNK)
                dstbv = lov - _splat(qbase)

                def emit(j, _):
                    v = j * 16 + iota
                    s = dstbv + v
                    plsc.store_scatter(
                        st_idx,
                        [lax.shift_right_logical(s, _splat(7)),
                         s & _splat(127)],
                        srcbv + v, mask=v < nv)
                    return 0
                lax.fori_loop(0, (n + 15) // 16, emit, 0)
            return 0
        lax.fori_loop(f_lo, f_hi, seg_body, 0)

        if last:
            for c in range(1, 8):
                st_idx[_GK // 128, pl.ds(c * 16, 16)] = zeros
        else:
            for c in range(2, 8):
                st_idx[_CHUNK // 128, pl.ds(c * 16, 16)] = zeros

        ngrp = 10 if last else 20

        def gat_body(gi, _):
            descs = []
            for u in range(8):
                r = gi * 8 + u
                descs.append(pltpu.async_copy(
                    h_p.at[st_idx.at[r]],
                    chunk_p.at[pl.ds(r * 128, 128)], sem))
                if not last:
                    descs.append(pltpu.async_copy(
                        h_k.at[st_idx.at[r]],
                        chunk_k.at[pl.ds(r * 128, 128)], sem))
            for dsc in descs:
                dsc.wait()
            return 0
        lax.fori_loop(0, ngrp, gat_body, 0)
        if not last:
            plsc.subcore_barrier()

    def extract_body(i, _):
        g = i * 16 + iota
        p = plsc.load_gather(chunk_p, [g])
        plsc.store_scatter(
            st_idx,
            [lax.shift_right_logical(g, _splat(7)), g & _splat(127)],
            p)
        return 0
    lax.fori_loop(0, _GK // 16, extract_body, 0)
    for c in range(1, 8):
        st_idx[_GK // 128, pl.ds(c * 16, 16)] = zeros

    obase = core * (2 * K) + t * _GK
    for ehbm, oo in ((src_hbm, 0), (dst_hbm, K)):
        def _fire_group(gi):
            descs = [pltpu.async_copy(ehbm.at[st_idx.at[gi * 8 + u]],
                                      tmprow.at[pl.ds(u * 128, 128)], sem)
                     for u in range(8)]
            for dsc in descs:
                dsc.wait()

        def out_body(gi, _):
            _fire_group(gi)
            pltpu.sync_copy(tmprow,
                            out_hbm.at[pl.ds(obase + oo + gi * 1024, 1024)])
            return 0
        lax.fori_loop(0, 9, out_body, 0)
        _fire_group(9)
        pltpu.sync_copy(tmprow.at[pl.ds(0, 784)],
                        out_hbm.at[pl.ds(obase + oo + 9216, 784)])


def _sc_topk(keys2, srcv, dstv):
    mesh = plsc.VectorSubcoreMesh(core_axis_name="c", subcore_axis_name="s",
                                  num_cores=2, num_subcores=_NT)
    f = pl.kernel(
        _sort_body,
        out_type=[
            jax.ShapeDtypeStruct((2 * 2 * K,), jnp.int32),
            jax.ShapeDtypeStruct((2 * E,), jnp.int32),
            jax.ShapeDtypeStruct((2 * E,), jnp.int32),
        ],
        mesh=mesh,
        compiler_params=pltpu.CompilerParams(needs_layout_passes=False),
        scratch_types=[
            pltpu.VMEM((20480,), jnp.int32),
            pltpu.VMEM((20480,), jnp.int32),
            pltpu.VMEM((160, 128), jnp.int32),
            pltpu.VMEM((_CHUNK,), jnp.int32),
            pltpu.VMEM((_CHUNK,), jnp.int32),
            pltpu.VMEM((1024,), jnp.int32),
            pltpu.VMEM((4096,), jnp.int32),
            pltpu.VMEM((4096,), jnp.int32),
            pltpu.VMEM((256,), jnp.int32),
            pltpu.VMEM((256,), jnp.int32),
            pltpu.VMEM((4096,), jnp.int32),
            pltpu.VMEM((4096,), jnp.int32),
            pltpu.VMEM((4096,), jnp.int32),
            pltpu.VMEM((2, 128), jnp.int32),
            pltpu.VMEM((16,), jnp.int32),
            pltpu.VMEM((16,), jnp.int32),
            pltpu.VMEM_SHARED((2 * 4096,), jnp.int32),
            pltpu.SemaphoreType.DMA,
        ],
    )
    return f(keys2, srcv, dstv)[0]


def _gat_ref(x, src, dst, dsts_sorted, perm, W, att_src, att_dst, b):
    n = x.shape[0]
    xp = x @ W
    a_src = (xp * att_src).sum(axis=-1)
    a_dst = (xp * att_dst).sum(axis=-1)
    alpha = jax.nn.leaky_relu(a_src[src] + a_dst[dst], negative_slope=0.2)
    amax = jax.ops.segment_max(alpha[perm], dsts_sorted, num_segments=n,
                               indices_are_sorted=True)
    amax = jnp.where(jnp.isfinite(amax), amax, 0.0)
    ex = jnp.exp(alpha - amax[dst])
    denom = jax.ops.segment_sum(ex, dst, num_segments=n)
    coef = ex / (denom[dst] + 1e-16)
    out = jax.ops.segment_sum(xp[src] * coef[:, None], dst, num_segments=n)
    return out + b


def _copy_kernel(x_ref, o_ref):
    o_ref[...] = x_ref[...]


def _pl_copy(x):
    return pl.pallas_call(
        _copy_kernel,
        out_shape=jax.ShapeDtypeStruct(x.shape, x.dtype),
    )(x)


def kernel(x, edge_index, W1, att_src1, att_dst1, b1, W2, att_src2, att_dst2, b2):
    x = _pl_copy(x)
    n = x.shape[0]
    loops = jnp.arange(n, dtype=edge_index.dtype)
    src = jnp.concatenate([edge_index[0], loops])
    dst = jnp.concatenate([edge_index[1], loops])
    dsts_sorted, perm = lax.sort(
        (dst, jnp.arange(dst.shape[0], dtype=jnp.int32)),
        num_keys=1, is_stable=True)
    xM1 = jax.nn.leaky_relu(
        _gat_ref(x, src, dst, dsts_sorted, perm, W1, att_src1, att_dst1, b1),
        negative_slope=0.01)
    xM2 = _gat_ref(xM1, src, dst, dsts_sorted, perm, W2, att_src2, att_dst2, b2)
    value = (xM2[edge_index[0]] * xM2[edge_index[1]]).sum(axis=1)

    bits = lax.bitcast_convert_type(value, jnp.uint32)
    sign = bits >> 31
    key_asc = bits ^ jnp.where(sign == 1, jnp.uint32(0xFFFFFFFF),
                               jnp.uint32(0x80000000))
    keys2 = jnp.concatenate([key_asc, ~key_asc]).astype(jnp.uint32)
    keys2 = lax.bitcast_convert_type(keys2, jnp.int32)

    out = _sc_topk(keys2, edge_index[0], edge_index[1]).reshape(2, 2, K)
    return (out[1], out[0], xM2)
